# Initial kernel scaffold; baseline (speedup 1.0000x reference)
import functools

import jax
import jax.numpy as jnp
from jax import lax
from jax.experimental import pallas as pl
from jax.experimental.pallas import tpu as pltpu
from jax.experimental.pallas import tpu_sc as plsc

N = 10000
D = 256
E = 160000
V = 8000

NC = 2
NS = 16
L = 16

HALF = N // NC
ACC_ROWS = 5120
TRASH = HALF
EPT = E // NS
CH = 80
NCHUNK = EPT // CH

DEG_ROWS = 10240
DEG_W = 16
EPS = E // NC
EPTD = EPS // NS
DCH = 40
NDC = EPTD // DCH

VPAD = 8192
BPW = VPAD // (NC * NS)

_sc_mesh = plsc.VectorSubcoreMesh(core_axis_name="c", subcore_axis_name="s")


@functools.partial(
    pl.kernel,
    out_type=jax.ShapeDtypeStruct((NC, DEG_ROWS, DEG_W), jnp.float32),
    mesh=_sc_mesh,
    scratch_types=[
        pltpu.VMEM((1, DCH), jnp.int32),
        pltpu.VMEM((DCH, DEG_W), jnp.float32),
        pltpu.VMEM_SHARED((DEG_ROWS, DEG_W), jnp.float32),
    ],
)
def _deg_kernel(dst_hbm, ones_hbm, zeros_hbm, out_hbm, didx, ones_v, acc):
    c = lax.axis_index("c")
    s = lax.axis_index("s")
    rpt = DEG_ROWS // NS
    pltpu.sync_copy(zeros_hbm, acc.at[pl.ds(s * rpt, rpt)])
    pltpu.sync_copy(ones_hbm, ones_v)
    plsc.subcore_barrier()
    ebase = c * EPS + s * EPTD

    def body(j, carry):
        off = pl.multiple_of(ebase + j * DCH, 8)
        pltpu.sync_copy(dst_hbm.at[pl.ds(off, DCH)], didx.at[0])
        pltpu.sync_copy(ones_v, acc.at[didx.at[0]], add=True)
        return carry

    lax.fori_loop(0, NDC, body, 0)
    plsc.subcore_barrier()
    pltpu.sync_copy(acc.at[pl.ds(s * rpt, rpt)],
                    out_hbm.at[c, pl.ds(s * rpt, rpt)])


@functools.partial(
    pl.kernel,
    out_type=jax.ShapeDtypeStruct((N, D), jnp.float32),
    mesh=_sc_mesh,
    scratch_types=[
        pltpu.VMEM((2, CH), jnp.int32),
        pltpu.VMEM((2, CH), jnp.int32),
        pltpu.VMEM((2, CH, D), jnp.float32),
        pltpu.VMEM_SHARED((ACC_ROWS, D), jnp.float32),
        pltpu.SemaphoreType.DMA,
        pltpu.SemaphoreType.DMA,
    ],
)
def _agg_kernel(src_hbm, dst_hbm, hs_hbm, zeros_hbm, out_hbm,
                sidx, didx, rows, acc, sem0, sem1):
    c = lax.axis_index("c")
    s = lax.axis_index("s")
    base_node = c * HALF
    pltpu.sync_copy(zeros_hbm,
                    acc.at[pl.ds(s * (ACC_ROWS // NS), ACC_ROWS // NS)])
    plsc.subcore_barrier()
    ebase = s * EPT
    sems = (sem0, sem1)

    def stage(j, b):
        off = pl.multiple_of(ebase + j * CH, 8)
        pltpu.sync_copy(src_hbm.at[pl.ds(off, CH)], sidx.at[b])
        pltpu.sync_copy(dst_hbm.at[pl.ds(off, CH)], didx.at[b])
        for v in range(CH // L):
            d = didx[b, pl.ds(v * L, L)]
            loc = d - base_node
            inb = (loc >= 0) & (loc < HALF)
            didx[b, pl.ds(v * L, L)] = jnp.where(inb, loc, TRASH)
        pltpu.async_copy(hs_hbm.at[sidx.at[b]], rows.at[b], sems[b])

    def drain_scatter(b):
        pltpu.make_async_copy(hs_hbm.at[sidx.at[b]], rows.at[b], sems[b]).wait()
        pltpu.sync_copy(rows.at[b], acc.at[didx.at[b]], add=True)

    stage(0, 0)

    def outer(k, carry):
        for b in range(2):
            j = 2 * k + b
            stage(j + 1, 1 - b)
            drain_scatter(b)
        return carry

    lax.fori_loop(0, (NCHUNK - 1) // 2, outer, 0)
    drain_scatter(0)
    plsc.subcore_barrier()
    woff = pl.multiple_of(base_node + s * 312, 8)
    pltpu.sync_copy(acc.at[pl.ds(s * 312, 312)], out_hbm.at[pl.ds(woff, 312)])

    @pl.when(s == NS - 1)
    def _tail():
        pltpu.sync_copy(
            acc.at[pl.ds(4992, 8)],
            out_hbm.at[pl.ds(pl.multiple_of(base_node + 4992, 8), 8)])


@functools.partial(
    pl.kernel,
    out_type=jax.ShapeDtypeStruct((VPAD, D), jnp.float32),
    mesh=_sc_mesh,
    scratch_types=[
        pltpu.VMEM((BPW,), jnp.int32),
        pltpu.VMEM((BPW, D), jnp.float32),
        pltpu.SemaphoreType.DMA,
    ],
)
def _gather_kernel(final_hbm, idx_hbm, out_hbm, idx_v, rows_v, sem):
    c = lax.axis_index("c")
    s = lax.axis_index("s")
    wid = s * NC + c
    base = wid * BPW
    pltpu.sync_copy(idx_hbm.at[pl.ds(base, BPW)], idx_v)
    for k in range(BPW // 128):
        pltpu.async_copy(final_hbm.at[idx_v.at[pl.ds(k * 128, 128)]],
                         rows_v.at[pl.ds(k * 128, 128)], sem).wait()
    pltpu.sync_copy(rows_v, out_hbm.at[pl.ds(base, BPW)])


RB = 1000


def _dinv(deg0_ref, deg1_ref):
    return lax.rsqrt(deg0_ref[:, 0:1] + deg1_ref[:, 0:1] + 1.0)


def _mm_scale_body(deg0_ref, deg1_ref, x_ref, w_ref, out_ref):
    out_ref[...] = jnp.dot(
        x_ref[...], w_ref[...],
        preferred_element_type=jnp.float32) * _dinv(deg0_ref, deg1_ref)


def _mid_body(deg0_ref, deg1_ref, agg_ref, hs_ref, b_ref, w_ref, out_ref):
    dinv = _dinv(deg0_ref, deg1_ref)
    x = jnp.maximum(dinv * (agg_ref[...] + hs_ref[...]) + b_ref[...], 0.0)
    out_ref[...] = jnp.dot(
        x, w_ref[...], preferred_element_type=jnp.float32) * dinv


def _final_body(deg0_ref, deg1_ref, agg_ref, hs_ref, b_ref, out_ref):
    dinv = _dinv(deg0_ref, deg1_ref)
    out_ref[...] = dinv * (agg_ref[...] + hs_ref[...]) + b_ref[...]


_deg_spec = pl.BlockSpec((RB, DEG_W), lambda i: (i, 0))
_row_spec = pl.BlockSpec((RB, D), lambda i: (i, 0))
_w_spec = pl.BlockSpec((D, D), lambda i: (0, 0))
_b_spec = pl.BlockSpec((1, D), lambda i: (0, 0))
_out_sds = jax.ShapeDtypeStruct((N, D), jnp.float32)

_mm_scale = pl.pallas_call(
    _mm_scale_body, grid=(N // RB,),
    in_specs=[_deg_spec, _deg_spec, _row_spec, _w_spec],
    out_specs=_row_spec, out_shape=_out_sds)

_mid = pl.pallas_call(
    _mid_body, grid=(N // RB,),
    in_specs=[_deg_spec, _deg_spec, _row_spec, _row_spec, _b_spec, _w_spec],
    out_specs=_row_spec, out_shape=_out_sds)

_final = pl.pallas_call(
    _final_body, grid=(N // RB,),
    in_specs=[_deg_spec, _deg_spec, _row_spec, _row_spec, _b_spec],
    out_specs=_row_spec, out_shape=_out_sds)


def kernel(embedding, edge_index, W1, b1, W2, b2, idx_mapping):
    src = edge_index[0]
    dst = edge_index[1]
    zeros_deg = jnp.zeros((DEG_ROWS // NS, DEG_W), jnp.float32)
    ones_deg = jnp.ones((DCH, DEG_W), jnp.float32)
    zeros_acc = jnp.zeros((ACC_ROWS // NS, D), jnp.float32)

    degp = _deg_kernel(dst, ones_deg, zeros_deg)
    deg0 = degp[0, :N]
    deg1 = degp[1, :N]

    hs1 = _mm_scale(deg0, deg1, embedding, W1)
    agg1 = _agg_kernel(src, dst, hs1, zeros_acc)
    hs2 = _mid(deg0, deg1, agg1, hs1, b1.reshape(1, D), W2)
    agg2 = _agg_kernel(src, dst, hs2, zeros_acc)
    finaln = _final(deg0, deg1, agg2, hs2, b2.reshape(1, D))

    idx_pad = jnp.concatenate(
        [idx_mapping, jnp.zeros((VPAD - V,), jnp.int32)])
    out = _gather_kernel(finaln, idx_pad)
    return out[:V]


# speedup vs baseline: 3.6699x; 3.6699x over previous
import functools

import jax
import jax.numpy as jnp
from jax import lax
from jax.experimental import pallas as pl
from jax.experimental.pallas import tpu as pltpu
from jax.experimental.pallas import tpu_sc as plsc

N = 10000
D = 256
E = 160000
V = 8000

NC = 2
NS = 16
L = 16
W = NC * NS

PT = 313
ACC_R = 320
TRASH_R = 316

SCH = 2000
NSC = E // SCH
FLUSH = 4096
BUFN = FLUSH + 32
DUMP = FLUSH + 16
CAP = 40 * FLUSH

CH = 80

VPAD = 8192
BPW = VPAD // W

_sc_mesh = plsc.VectorSubcoreMesh(core_axis_name="c", subcore_axis_name="s")
_sc_params = pltpu.CompilerParams(needs_layout_passes=False)


def _wid():
    return lax.axis_index("s") * NC + lax.axis_index("c")


@functools.partial(
    pl.kernel,
    out_type=[
        jax.ShapeDtypeStruct((W * CAP,), jnp.int32),
        jax.ShapeDtypeStruct((W * CAP,), jnp.int32),
        jax.ShapeDtypeStruct((W * L,), jnp.int32),
    ],
    mesh=_sc_mesh,
    compiler_params=_sc_params,
    scratch_types=[
        pltpu.VMEM((SCH,), jnp.int32),
        pltpu.VMEM((SCH,), jnp.int32),
        pltpu.VMEM((BUFN,), jnp.int32),
        pltpu.VMEM((BUFN,), jnp.int32),
        pltpu.VMEM((L,), jnp.int32),
    ],
)
def _route_kernel(src_hbm, dst_hbm, esrc_hbm, edloc_hbm, cnt_hbm,
                  sstage, dstage, bufs, bufd, cntv):
    w = _wid()
    lo = w * PT
    wbase = w * CAP

    def flush(state):
        cnt, flushed = state
        hit = cnt >= FLUSH

        @pl.when(hit)
        def _():
            off = pl.multiple_of(wbase + flushed * FLUSH, 8)
            pltpu.sync_copy(bufs.at[pl.ds(0, FLUSH)],
                            esrc_hbm.at[pl.ds(off, FLUSH)])
            pltpu.sync_copy(bufd.at[pl.ds(0, FLUSH)],
                            edloc_hbm.at[pl.ds(off, FLUSH)])
            bufs[pl.ds(0, L)] = bufs[pl.ds(FLUSH, L)]
            bufd[pl.ds(0, L)] = bufd[pl.ds(FLUSH, L)]

        return (jnp.where(hit, cnt - FLUSH, cnt),
                jnp.where(hit, flushed + 1, flushed))

    def scan_chunk(ch, state):
        off = pl.multiple_of(ch * SCH, 8)
        pltpu.sync_copy(src_hbm.at[pl.ds(off, SCH)], sstage)
        pltpu.sync_copy(dst_hbm.at[pl.ds(off, SCH)], dstage)

        def group(g, st):
            cnt, flushed = st
            dvec = dstage[pl.ds(g * L, L)]
            svec = sstage[pl.ds(g * L, L)]
            loc = dvec - lo
            inb = (loc >= 0) & (loc < PT)
            pos = cnt + plsc.cumsum(inb.astype(jnp.int32)) - 1
            pos = jnp.where(inb, pos, DUMP)
            plsc.store_scatter(bufs, [pos], svec)
            plsc.store_scatter(bufd, [pos], loc)
            cnt = cnt + plsc.all_reduce_population_count(inb)[0]
            return flush((cnt, flushed))

        return lax.fori_loop(0, SCH // L, group, state, unroll=False)

    cnt, flushed = lax.fori_loop(0, NSC, scan_chunk, (0, 0), unroll=False)
    off = pl.multiple_of(wbase + flushed * FLUSH, 8)
    pltpu.sync_copy(bufs.at[pl.ds(0, FLUSH)], esrc_hbm.at[pl.ds(off, FLUSH)])
    pltpu.sync_copy(bufd.at[pl.ds(0, FLUSH)], edloc_hbm.at[pl.ds(off, FLUSH)])
    total = flushed * FLUSH + cnt
    cntv[pl.ds(0, L)] = jnp.full((L,), total, dtype=jnp.int32)
    pltpu.sync_copy(cntv, cnt_hbm.at[pl.ds(pl.multiple_of(w * L, 8), L)])


@functools.partial(
    pl.kernel,
    out_type=jax.ShapeDtypeStruct((W * ACC_R, L), jnp.float32),
    mesh=_sc_mesh,
    scratch_types=[
        pltpu.VMEM((CH,), jnp.int32),
        pltpu.VMEM((ACC_R, L), jnp.float32),
        pltpu.VMEM((L,), jnp.int32),
    ],
)
def _deg_kernel(edloc_hbm, cnt_hbm, deg_hbm, didx, dacc, cntv):
    w = _wid()
    wbase = w * CAP
    pltpu.sync_copy(cnt_hbm.at[pl.ds(pl.multiple_of(w * L, 8), L)], cntv)
    total = cntv[pl.ds(0, L)][0]

    def zrow(r, carry):
        dacc[r, pl.ds(0, L)] = jnp.zeros((L,), jnp.float32)
        return carry

    lax.fori_loop(0, ACC_R, zrow, 0, unroll=False)
    ones = jnp.ones((L,), jnp.float32)
    iota = lax.iota(jnp.int32, L)

    def replay(j, carry):
        off = pl.multiple_of(wbase + j * CH, 8)
        pltpu.sync_copy(edloc_hbm.at[pl.ds(off, CH)], didx)
        for g in range(CH // L):
            dvec = didx[pl.ds(g * L, L)]
            lane = j * CH + g * L + iota
            ok = (lane < total) & (dvec >= 0) & (dvec < PT)
            dv = jnp.where(ok, dvec, TRASH_R)
            for l in range(L):
                plsc.addupdate(dacc.at[dv[l], pl.ds(0, L)], ones)
        return carry

    lax.fori_loop(0, (total + CH - 1) // CH, replay, 0, unroll=False)
    pltpu.sync_copy(dacc, deg_hbm.at[pl.ds(w * ACC_R, ACC_R)])


@functools.partial(
    pl.kernel,
    out_type=jax.ShapeDtypeStruct((W * ACC_R, D), jnp.float32),
    mesh=_sc_mesh,
    scratch_types=[
        pltpu.VMEM((CH,), jnp.int32),
        pltpu.VMEM((CH,), jnp.int32),
        pltpu.VMEM((CH, D), jnp.float32),
        pltpu.VMEM((ACC_R, D), jnp.float32),
        pltpu.VMEM((L,), jnp.int32),
        pltpu.SemaphoreType.DMA,
    ],
)
def _agg_kernel(esrc_hbm, edloc_hbm, cnt_hbm, hs_hbm, zeros_hbm, out_hbm,
                sidx, didx, rows, acc, cntv, sem):
    w = _wid()
    wbase = w * CAP
    pltpu.sync_copy(zeros_hbm, acc)
    pltpu.sync_copy(cnt_hbm.at[pl.ds(pl.multiple_of(w * L, 8), L)], cntv)
    total = cntv[pl.ds(0, L)][0]
    iota = lax.iota(jnp.int32, L)

    def chunk(j, carry):
        off = pl.multiple_of(wbase + j * CH, 8)
        pltpu.sync_copy(esrc_hbm.at[pl.ds(off, CH)], sidx)
        pltpu.sync_copy(edloc_hbm.at[pl.ds(off, CH)], didx)
        for g in range(CH // L):
            sv = sidx[pl.ds(g * L, L)]
            sv = jnp.where(sv < 0, 0, sv)
            sidx[pl.ds(g * L, L)] = jnp.where(sv >= N, 0, sv)
        pltpu.async_copy(hs_hbm.at[sidx], rows, sem).wait()
        for g in range(CH // L):
            dvec = didx[pl.ds(g * L, L)]
            lane = j * CH + g * L + iota
            ok = (lane < total) & (dvec >= 0) & (dvec < PT)
            dv = jnp.where(ok, dvec, TRASH_R)
            for l in range(L):
                e = g * L + l
                for k in range(D // L):
                    plsc.addupdate(acc.at[dv[l], pl.ds(k * L, L)],
                                   rows[e, pl.ds(k * L, L)])
        return carry

    lax.fori_loop(0, (total + CH - 1) // CH, chunk, 0, unroll=False)
    pltpu.sync_copy(acc, out_hbm.at[pl.ds(w * ACC_R, ACC_R)])


@functools.partial(
    pl.kernel,
    out_type=jax.ShapeDtypeStruct((VPAD, D), jnp.float32),
    mesh=_sc_mesh,
    scratch_types=[
        pltpu.VMEM((BPW,), jnp.int32),
        pltpu.VMEM((BPW, D), jnp.float32),
        pltpu.SemaphoreType.DMA,
    ],
)
def _gather_kernel(final_hbm, idx_hbm, out_hbm, idx_v, rows_v, sem):
    base = _wid() * BPW
    pltpu.sync_copy(idx_hbm.at[pl.ds(base, BPW)], idx_v)
    for k in range(BPW // 128):
        pltpu.async_copy(final_hbm.at[idx_v.at[pl.ds(k * 128, 128)]],
                         rows_v.at[pl.ds(k * 128, 128)], sem).wait()
    pltpu.sync_copy(rows_v, out_hbm.at[pl.ds(base, BPW)])


RB = 1000


def _dinv(deg_ref):
    return lax.rsqrt(deg_ref[:, 0:1] + 1.0)


def _mm_scale_body(deg_ref, x_ref, w_ref, out_ref):
    out_ref[...] = jnp.dot(
        x_ref[...], w_ref[...],
        preferred_element_type=jnp.float32) * _dinv(deg_ref)


def _mid_body(deg_ref, agg_ref, hs_ref, b_ref, w_ref, out_ref):
    dinv = _dinv(deg_ref)
    x = jnp.maximum(dinv * (agg_ref[...] + hs_ref[...]) + b_ref[...], 0.0)
    out_ref[...] = jnp.dot(
        x, w_ref[...], preferred_element_type=jnp.float32) * dinv


def _final_body(deg_ref, agg_ref, hs_ref, b_ref, out_ref):
    dinv = _dinv(deg_ref)
    out_ref[...] = dinv * (agg_ref[...] + hs_ref[...]) + b_ref[...]


_deg_spec = pl.BlockSpec((RB, L), lambda i: (i, 0))
_row_spec = pl.BlockSpec((RB, D), lambda i: (i, 0))
_w_spec = pl.BlockSpec((D, D), lambda i: (0, 0))
_b_spec = pl.BlockSpec((1, D), lambda i: (0, 0))
_out_sds = jax.ShapeDtypeStruct((N, D), jnp.float32)

_mm_scale = pl.pallas_call(
    _mm_scale_body, grid=(N // RB,),
    in_specs=[_deg_spec, _row_spec, _w_spec],
    out_specs=_row_spec, out_shape=_out_sds)

_mid = pl.pallas_call(
    _mid_body, grid=(N // RB,),
    in_specs=[_deg_spec, _row_spec, _row_spec, _b_spec, _w_spec],
    out_specs=_row_spec, out_shape=_out_sds)

_final = pl.pallas_call(
    _final_body, grid=(N // RB,),
    in_specs=[_deg_spec, _row_spec, _row_spec, _b_spec],
    out_specs=_row_spec, out_shape=_out_sds)


def _unzone(x):
    dcols = x.shape[-1]
    return x.reshape(W, ACC_R, dcols)[:, :PT].reshape(W * PT, dcols)[:N]


def kernel(embedding, edge_index, W1, b1, W2, b2, idx_mapping):
    src = edge_index[0]
    dst = edge_index[1]
    zeros_agg = jnp.zeros((ACC_R, D), jnp.float32)

    esrc, edloc, counts = _route_kernel(src, dst)
    deg = _unzone(_deg_kernel(edloc, counts))

    hs1 = _mm_scale(deg, embedding, W1)
    agg1 = _unzone(_agg_kernel(esrc, edloc, counts, hs1, zeros_agg))
    hs2 = _mid(deg, agg1, hs1, b1.reshape(1, D), W2)
    agg2 = _unzone(_agg_kernel(esrc, edloc, counts, hs2, zeros_agg))
    finaln = _final(deg, agg2, hs2, b2.reshape(1, D))

    idx_pad = jnp.concatenate(
        [idx_mapping, jnp.zeros((VPAD - V,), jnp.int32)])
    out = _gather_kernel(finaln, idx_pad)
    return out[:V]


# speedup vs baseline: 5.5474x; 1.5116x over previous
import functools

import jax
import jax.numpy as jnp
from jax import lax
from jax.experimental import pallas as pl
from jax.experimental.pallas import tpu as pltpu
from jax.experimental.pallas import tpu_sc as plsc

N = 10000
D = 256
E = 160000
V = 8000

NC = 2
NS = 16
L = 16
W = NC * NS

PT = 313
ACC_R = 320
TRASH_R = 316

SCH = 4000
NSC = E // SCH
FLUSH = 4096
BUFN = FLUSH + 32
DUMP = FLUSH + 16
CAP = 40 * FLUSH
CAPT = CAP + 256

CH = 80

VPAD = 8192
BPW = VPAD // W

_sc_mesh = plsc.VectorSubcoreMesh(core_axis_name="c", subcore_axis_name="s")
_sc_params = pltpu.CompilerParams(needs_layout_passes=False)


def _wid():
    return lax.axis_index("s") * NC + lax.axis_index("c")


@functools.partial(
    pl.kernel,
    out_type=[
        jax.ShapeDtypeStruct((W * CAPT,), jnp.int32),
        jax.ShapeDtypeStruct((W * CAPT,), jnp.int32),
        jax.ShapeDtypeStruct((W * L,), jnp.int32),
    ],
    mesh=_sc_mesh,
    compiler_params=_sc_params,
    scratch_types=[
        pltpu.VMEM((SCH,), jnp.int32),
        pltpu.VMEM((SCH,), jnp.int32),
        pltpu.VMEM((SCH,), jnp.int32),
        pltpu.VMEM((SCH,), jnp.int32),
        pltpu.VMEM((BUFN,), jnp.int32),
        pltpu.VMEM((BUFN,), jnp.int32),
        pltpu.VMEM((L,), jnp.int32),
        pltpu.SemaphoreType.DMA,
        pltpu.SemaphoreType.DMA,
    ],
)
def _route_kernel(src_hbm, dst_hbm, esrc_hbm, edloc_hbm, cnt_hbm,
                  sstage0, sstage1, dstage0, dstage1, bufs, bufd, cntv,
                  ssem0, ssem1):
    w = _wid()
    lo = w * PT
    wbase = w * CAPT
    sstages = (sstage0, sstage1)
    dstages = (dstage0, dstage1)
    ssems = (ssem0, ssem1)

    def flush(state):
        cnt, flushed = state
        hit = cnt >= FLUSH

        @pl.when(hit)
        def _():
            off = pl.multiple_of(wbase + flushed * FLUSH, 8)
            pltpu.sync_copy(bufs.at[pl.ds(0, FLUSH)],
                            esrc_hbm.at[pl.ds(off, FLUSH)])
            pltpu.sync_copy(bufd.at[pl.ds(0, FLUSH)],
                            edloc_hbm.at[pl.ds(off, FLUSH)])
            bufs[pl.ds(0, L)] = bufs[pl.ds(FLUSH, L)]
            bufd[pl.ds(0, L)] = bufd[pl.ds(FLUSH, L)]

        return (jnp.where(hit, cnt - FLUSH, cnt),
                jnp.where(hit, flushed + 1, flushed))

    def stage_async(ch, b):
        off = pl.multiple_of(ch * SCH, 8)
        pltpu.async_copy(src_hbm.at[pl.ds(off, SCH)], sstages[b], ssems[b])
        pltpu.async_copy(dst_hbm.at[pl.ds(off, SCH)], dstages[b], ssems[b])

    def wait_stage(b):
        pltpu.make_async_copy(src_hbm.at[pl.ds(0, SCH)], sstages[b],
                              ssems[b]).wait()
        pltpu.make_async_copy(dst_hbm.at[pl.ds(0, SCH)], dstages[b],
                              ssems[b]).wait()

    def scan_chunk(ch, b, state):
        wait_stage(b)

        @pl.when(ch + 1 < NSC)
        def _():
            stage_async(ch + 1, 1 - b)

        def group(g, st):
            cnt, flushed = st
            dvec = dstages[b][pl.ds(g * L, L)]
            svec = sstages[b][pl.ds(g * L, L)]
            loc = dvec - lo
            inb = (loc >= 0) & (loc < PT)
            pos = cnt + plsc.cumsum(inb.astype(jnp.int32)) - 1
            pos = jnp.where(inb, pos, DUMP)
            plsc.store_scatter(bufs, [pos], svec)
            plsc.store_scatter(bufd, [pos], loc)
            cnt = cnt + plsc.all_reduce_population_count(inb)[0]
            return flush((cnt, flushed))

        return lax.fori_loop(0, SCH // L, group, state, unroll=False)

    stage_async(0, 0)

    def scan_pair(k, state):
        for b in range(2):
            state = scan_chunk(2 * k + b, b, state)
        return state

    cnt, flushed = lax.fori_loop(0, NSC // 2, scan_pair, (0, 0),
                                 unroll=False)
    off = pl.multiple_of(wbase + flushed * FLUSH, 8)
    pltpu.sync_copy(bufs.at[pl.ds(0, FLUSH)], esrc_hbm.at[pl.ds(off, FLUSH)])
    pltpu.sync_copy(bufd.at[pl.ds(0, FLUSH)], edloc_hbm.at[pl.ds(off, FLUSH)])
    total = flushed * FLUSH + cnt
    cntv[pl.ds(0, L)] = jnp.full((L,), total, dtype=jnp.int32)
    pltpu.sync_copy(cntv, cnt_hbm.at[pl.ds(pl.multiple_of(w * L, 8), L)])


@functools.partial(
    pl.kernel,
    out_type=jax.ShapeDtypeStruct((W * ACC_R, L), jnp.float32),
    mesh=_sc_mesh,
    scratch_types=[
        pltpu.VMEM((CH,), jnp.int32),
        pltpu.VMEM((ACC_R, L), jnp.float32),
        pltpu.VMEM((L,), jnp.int32),
    ],
)
def _deg_kernel(edloc_hbm, cnt_hbm, deg_hbm, didx, dacc, cntv):
    w = _wid()
    wbase = w * CAPT
    pltpu.sync_copy(cnt_hbm.at[pl.ds(pl.multiple_of(w * L, 8), L)], cntv)
    total = cntv[pl.ds(0, L)][0]

    def zrow(r, carry):
        dacc[r, pl.ds(0, L)] = jnp.zeros((L,), jnp.float32)
        return carry

    lax.fori_loop(0, ACC_R, zrow, 0, unroll=False)
    ones = jnp.ones((L,), jnp.float32)
    iota = lax.iota(jnp.int32, L)

    def replay(j, carry):
        off = pl.multiple_of(wbase + j * CH, 8)
        pltpu.sync_copy(edloc_hbm.at[pl.ds(off, CH)], didx)
        for g in range(CH // L):
            dvec = didx[pl.ds(g * L, L)]
            lane = j * CH + g * L + iota
            ok = (lane < total) & (dvec >= 0) & (dvec < PT)
            dv = jnp.where(ok, dvec, TRASH_R)
            for l in range(L):
                plsc.addupdate(dacc.at[dv[l], pl.ds(0, L)], ones)
        return carry

    lax.fori_loop(0, (total + CH - 1) // CH, replay, 0, unroll=False)
    pltpu.sync_copy(dacc, deg_hbm.at[pl.ds(w * ACC_R, ACC_R)])


@functools.partial(
    pl.kernel,
    out_type=jax.ShapeDtypeStruct((W * ACC_R, D), jnp.float32),
    mesh=_sc_mesh,
    scratch_types=[
        pltpu.VMEM((CH,), jnp.int32),
        pltpu.VMEM((CH,), jnp.int32),
        pltpu.VMEM((CH,), jnp.int32),
        pltpu.VMEM((CH,), jnp.int32),
        pltpu.VMEM((CH,), jnp.int32),
        pltpu.VMEM((2, CH, D), jnp.float32),
        pltpu.VMEM((ACC_R, D), jnp.float32),
        pltpu.VMEM((L,), jnp.int32),
        pltpu.SemaphoreType.DMA,
        pltpu.SemaphoreType.DMA,
        pltpu.SemaphoreType.DMA,
        pltpu.SemaphoreType.DMA,
    ],
)
def _agg_kernel(esrc_hbm, edloc_hbm, cnt_hbm, hs_hbm, zeros_hbm, out_hbm,
                sidx0, sidx1, didx0, didx1, dtmp, rows, acc, cntv,
                gsem0, gsem1, ssem0, ssem1):
    w = _wid()
    wbase = w * CAPT
    pltpu.sync_copy(zeros_hbm, acc)
    pltpu.sync_copy(cnt_hbm.at[pl.ds(pl.multiple_of(w * L, 8), L)], cntv)
    total = cntv[pl.ds(0, L)][0]
    iota = lax.iota(jnp.int32, L)
    sidxs = (sidx0, sidx1)
    didxs = (didx0, didx1)
    gsems = (gsem0, gsem1)
    ssems = (ssem0, ssem1)

    def stage_async(j, b):
        off = pl.multiple_of(wbase + j * CH, 8)
        pltpu.async_copy(esrc_hbm.at[pl.ds(off, CH)], sidxs[b], ssems[b])
        pltpu.async_copy(edloc_hbm.at[pl.ds(off, CH)], didxs[b], ssems[b])

    def wait_stage(b):
        pltpu.make_async_copy(esrc_hbm.at[pl.ds(0, CH)], sidxs[b],
                              ssems[b]).wait()
        pltpu.make_async_copy(edloc_hbm.at[pl.ds(0, CH)], didxs[b],
                              ssems[b]).wait()

    def clamp_launch_gather(b):
        for g in range(CH // L):
            sv = sidxs[b][pl.ds(g * L, L)]
            sv = jnp.where(sv < 0, 0, sv)
            sidxs[b][pl.ds(g * L, L)] = jnp.where(sv >= N, 0, sv)
        pltpu.async_copy(hs_hbm.at[sidxs[b]], rows.at[b], gsems[b])

    def wait_gather(b):
        pltpu.make_async_copy(hs_hbm.at[sidxs[b]], rows.at[b],
                              gsems[b]).wait()

    stage_async(0, 0)
    wait_stage(0)
    clamp_launch_gather(0)
    stage_async(1, 1)

    M = jnp.maximum((total + CH - 1) // CH, 1)

    def pair(k, carry):
        for b in range(2):
            j = 2 * k + b
            b2 = 1 - b
            wait_gather(b)
            for g in range(CH // L):
                dtmp[pl.ds(g * L, L)] = didxs[b][pl.ds(g * L, L)]
            wait_stage(b2)
            clamp_launch_gather(b2)
            stage_async(j + 2, b)

            def group(g, c2):
                dvec = dtmp[pl.ds(g * L, L)]
                lane = j * CH + g * L + iota
                ok = (lane < total) & (dvec >= 0) & (dvec < PT)
                dv = jnp.where(ok, dvec, TRASH_R)
                for l in range(L):
                    for kk in range(D // L):
                        plsc.addupdate(
                            acc.at[dv[l], pl.ds(kk * L, L)],
                            rows[b, g * L + l, pl.ds(kk * L, L)])
                return c2

            lax.fori_loop(0, CH // L, group, 0, unroll=False)
        return carry

    lax.fori_loop(0, (M + 1) // 2, pair, 0, unroll=False)
    wait_gather(0)
    wait_stage(1)
    pltpu.sync_copy(acc, out_hbm.at[pl.ds(w * ACC_R, ACC_R)])


@functools.partial(
    pl.kernel,
    out_type=jax.ShapeDtypeStruct((VPAD, D), jnp.float32),
    mesh=_sc_mesh,
    scratch_types=[
        pltpu.VMEM((BPW,), jnp.int32),
        pltpu.VMEM((BPW, D), jnp.float32),
        pltpu.SemaphoreType.DMA,
    ],
)
def _gather_kernel(final_hbm, idx_hbm, out_hbm, idx_v, rows_v, sem):
    base = _wid() * BPW
    pltpu.sync_copy(idx_hbm.at[pl.ds(base, BPW)], idx_v)
    for k in range(BPW // 128):
        pltpu.async_copy(final_hbm.at[idx_v.at[pl.ds(k * 128, 128)]],
                         rows_v.at[pl.ds(k * 128, 128)], sem).wait()
    pltpu.sync_copy(rows_v, out_hbm.at[pl.ds(base, BPW)])


RB = 1000


def _dinv(deg_ref):
    return lax.rsqrt(deg_ref[:, 0:1] + 1.0)


def _mm_scale_body(deg_ref, x_ref, w_ref, out_ref):
    out_ref[...] = jnp.dot(
        x_ref[...], w_ref[...],
        preferred_element_type=jnp.float32) * _dinv(deg_ref)


def _mid_body(deg_ref, agg_ref, hs_ref, b_ref, w_ref, out_ref):
    dinv = _dinv(deg_ref)
    x = jnp.maximum(dinv * (agg_ref[...] + hs_ref[...]) + b_ref[...], 0.0)
    out_ref[...] = jnp.dot(
        x, w_ref[...], preferred_element_type=jnp.float32) * dinv


def _final_body(deg_ref, agg_ref, hs_ref, b_ref, out_ref):
    dinv = _dinv(deg_ref)
    out_ref[...] = dinv * (agg_ref[...] + hs_ref[...]) + b_ref[...]


_deg_spec = pl.BlockSpec((RB, L), lambda i: (i, 0))
_row_spec = pl.BlockSpec((RB, D), lambda i: (i, 0))
_w_spec = pl.BlockSpec((D, D), lambda i: (0, 0))
_b_spec = pl.BlockSpec((1, D), lambda i: (0, 0))
_out_sds = jax.ShapeDtypeStruct((N, D), jnp.float32)

_mm_scale = pl.pallas_call(
    _mm_scale_body, grid=(N // RB,),
    in_specs=[_deg_spec, _row_spec, _w_spec],
    out_specs=_row_spec, out_shape=_out_sds)

_mid = pl.pallas_call(
    _mid_body, grid=(N // RB,),
    in_specs=[_deg_spec, _row_spec, _row_spec, _b_spec, _w_spec],
    out_specs=_row_spec, out_shape=_out_sds)

_final = pl.pallas_call(
    _final_body, grid=(N // RB,),
    in_specs=[_deg_spec, _row_spec, _row_spec, _b_spec],
    out_specs=_row_spec, out_shape=_out_sds)


def _unzone(x):
    dcols = x.shape[-1]
    return x.reshape(W, ACC_R, dcols)[:, :PT].reshape(W * PT, dcols)[:N]


def kernel(embedding, edge_index, W1, b1, W2, b2, idx_mapping):
    src = edge_index[0]
    dst = edge_index[1]
    zeros_agg = jnp.zeros((ACC_R, D), jnp.float32)

    esrc, edloc, counts = _route_kernel(src, dst)
    deg = _unzone(_deg_kernel(edloc, counts))

    hs1 = _mm_scale(deg, embedding, W1)
    agg1 = _unzone(_agg_kernel(esrc, edloc, counts, hs1, zeros_agg))
    hs2 = _mid(deg, agg1, hs1, b1.reshape(1, D), W2)
    agg2 = _unzone(_agg_kernel(esrc, edloc, counts, hs2, zeros_agg))
    finaln = _final(deg, agg2, hs2, b2.reshape(1, D))

    idx_pad = jnp.concatenate(
        [idx_mapping, jnp.zeros((VPAD - V,), jnp.int32)])
    out = _gather_kernel(finaln, idx_pad)
    return out[:V]


# speedup vs baseline: 6.4431x; 1.1615x over previous
import functools

import jax
import jax.numpy as jnp
from jax import lax
from jax.experimental import pallas as pl
from jax.experimental.pallas import tpu as pltpu
from jax.experimental.pallas import tpu_sc as plsc

N = 10000
D = 256
E = 160000
V = 8000

NC = 2
NS = 16
L = 16
W = NC * NS

PT = 313
ACC_R = 320
TRASH_R = 316

SCH = 4000
NSC = E // SCH
FLUSH = 4096
BUFN = FLUSH + 32
DUMP = FLUSH + 16
CAP = 40 * FLUSH
CAPT = CAP + 256

CH = 80

VPAD = 8192
BPW = VPAD // W

_sc_mesh = plsc.VectorSubcoreMesh(core_axis_name="c", subcore_axis_name="s")
_sc_params = pltpu.CompilerParams(needs_layout_passes=False)


def _wid():
    return lax.axis_index("s") * NC + lax.axis_index("c")


@functools.partial(
    pl.kernel,
    out_type=[
        jax.ShapeDtypeStruct((W * CAPT,), jnp.int32),
        jax.ShapeDtypeStruct((W * CAPT,), jnp.int32),
        jax.ShapeDtypeStruct((W * L,), jnp.int32),
    ],
    mesh=_sc_mesh,
    compiler_params=_sc_params,
    scratch_types=[
        pltpu.VMEM((SCH,), jnp.int32),
        pltpu.VMEM((SCH,), jnp.int32),
        pltpu.VMEM((SCH,), jnp.int32),
        pltpu.VMEM((SCH,), jnp.int32),
        pltpu.VMEM((BUFN,), jnp.int32),
        pltpu.VMEM((BUFN,), jnp.int32),
        pltpu.VMEM((L,), jnp.int32),
        pltpu.SemaphoreType.DMA,
        pltpu.SemaphoreType.DMA,
    ],
)
def _route_kernel(src_hbm, dst_hbm, esrc_hbm, edloc_hbm, cnt_hbm,
                  sstage0, sstage1, dstage0, dstage1, bufs, bufd, cntv,
                  ssem0, ssem1):
    w = _wid()
    lo = w * PT
    wbase = w * CAPT
    sstages = (sstage0, sstage1)
    dstages = (dstage0, dstage1)
    ssems = (ssem0, ssem1)

    def flush(state):
        cnt, flushed = state
        hit = cnt >= FLUSH

        @pl.when(hit)
        def _():
            off = pl.multiple_of(wbase + flushed * FLUSH, 8)
            pltpu.sync_copy(bufs.at[pl.ds(0, FLUSH)],
                            esrc_hbm.at[pl.ds(off, FLUSH)])
            pltpu.sync_copy(bufd.at[pl.ds(0, FLUSH)],
                            edloc_hbm.at[pl.ds(off, FLUSH)])
            bufs[pl.ds(0, L)] = bufs[pl.ds(FLUSH, L)]
            bufd[pl.ds(0, L)] = bufd[pl.ds(FLUSH, L)]

        return (jnp.where(hit, cnt - FLUSH, cnt),
                jnp.where(hit, flushed + 1, flushed))

    def stage_async(ch, b):
        off = pl.multiple_of(ch * SCH, 8)
        pltpu.async_copy(src_hbm.at[pl.ds(off, SCH)], sstages[b], ssems[b])
        pltpu.async_copy(dst_hbm.at[pl.ds(off, SCH)], dstages[b], ssems[b])

    def wait_stage(b):
        pltpu.make_async_copy(src_hbm.at[pl.ds(0, SCH)], sstages[b],
                              ssems[b]).wait()
        pltpu.make_async_copy(dst_hbm.at[pl.ds(0, SCH)], dstages[b],
                              ssems[b]).wait()

    def scan_chunk(ch, b, state):
        wait_stage(b)

        @pl.when(ch + 1 < NSC)
        def _():
            stage_async(ch + 1, 1 - b)

        def group(g, st):
            cnt, flushed = st
            dvec = dstages[b][pl.ds(g * L, L)]
            svec = sstages[b][pl.ds(g * L, L)]
            loc = dvec - lo
            inb = (loc >= 0) & (loc < PT)
            cs = plsc.cumsum(inb.astype(jnp.int32))
            pos = jnp.where(inb, cnt + cs - 1, DUMP)
            plsc.store_scatter(bufs, [pos], svec)
            plsc.store_scatter(bufd, [pos], loc)
            cnt = cnt + cs[L - 1]
            return flush((cnt, flushed))

        return lax.fori_loop(0, SCH // L, group, state, unroll=False)

    stage_async(0, 0)

    def scan_pair(k, state):
        for b in range(2):
            state = scan_chunk(2 * k + b, b, state)
        return state

    cnt, flushed = lax.fori_loop(0, NSC // 2, scan_pair, (0, 0),
                                 unroll=False)
    off = pl.multiple_of(wbase + flushed * FLUSH, 8)
    pltpu.sync_copy(bufs.at[pl.ds(0, FLUSH)], esrc_hbm.at[pl.ds(off, FLUSH)])
    pltpu.sync_copy(bufd.at[pl.ds(0, FLUSH)], edloc_hbm.at[pl.ds(off, FLUSH)])
    total = flushed * FLUSH + cnt
    cntv[pl.ds(0, L)] = jnp.full((L,), total, dtype=jnp.int32)
    pltpu.sync_copy(cntv, cnt_hbm.at[pl.ds(pl.multiple_of(w * L, 8), L)])


@functools.partial(
    pl.kernel,
    out_type=jax.ShapeDtypeStruct((W * ACC_R, L), jnp.float32),
    mesh=_sc_mesh,
    scratch_types=[
        pltpu.VMEM((CH,), jnp.int32),
        pltpu.VMEM((ACC_R, L), jnp.float32),
        pltpu.VMEM((L,), jnp.int32),
    ],
)
def _deg_kernel(edloc_hbm, cnt_hbm, deg_hbm, didx, dacc, cntv):
    w = _wid()
    wbase = w * CAPT
    pltpu.sync_copy(cnt_hbm.at[pl.ds(pl.multiple_of(w * L, 8), L)], cntv)
    total = cntv[pl.ds(0, L)][0]

    def zrow(r, carry):
        dacc[r, pl.ds(0, L)] = jnp.zeros((L,), jnp.float32)
        return carry

    lax.fori_loop(0, ACC_R, zrow, 0, unroll=False)
    ones = jnp.ones((L,), jnp.float32)
    iota = lax.iota(jnp.int32, L)

    def replay(j, carry):
        off = pl.multiple_of(wbase + j * CH, 8)
        pltpu.sync_copy(edloc_hbm.at[pl.ds(off, CH)], didx)
        for g in range(CH // L):
            dvec = didx[pl.ds(g * L, L)]
            lane = j * CH + g * L + iota
            ok = (lane < total) & (dvec >= 0) & (dvec < PT)
            dv = jnp.where(ok, dvec, TRASH_R)
            for l in range(L):
                plsc.addupdate(dacc.at[dv[l], pl.ds(0, L)], ones)
        return carry

    lax.fori_loop(0, (total + CH - 1) // CH, replay, 0, unroll=False)
    pltpu.sync_copy(dacc, deg_hbm.at[pl.ds(w * ACC_R, ACC_R)])


@functools.partial(
    pl.kernel,
    out_type=jax.ShapeDtypeStruct((W * ACC_R, D), jnp.float32),
    mesh=_sc_mesh,
    scratch_types=[
        pltpu.VMEM((CH,), jnp.int32),
        pltpu.VMEM((CH,), jnp.int32),
        pltpu.VMEM((CH,), jnp.int32),
        pltpu.VMEM((CH,), jnp.int32),
        pltpu.VMEM((CH,), jnp.int32),
        pltpu.VMEM((2, CH, D), jnp.float32),
        pltpu.VMEM((ACC_R, D), jnp.float32),
        pltpu.VMEM((L,), jnp.int32),
        pltpu.SemaphoreType.DMA,
        pltpu.SemaphoreType.DMA,
        pltpu.SemaphoreType.DMA,
        pltpu.SemaphoreType.DMA,
    ],
)
def _agg_kernel(esrc_hbm, edloc_hbm, cnt_hbm, hs_hbm, zeros_hbm, out_hbm,
                sidx0, sidx1, didx0, didx1, dtmp, rows, acc, cntv,
                gsem0, gsem1, ssem0, ssem1):
    w = _wid()
    wbase = w * CAPT
    pltpu.sync_copy(zeros_hbm, acc)
    pltpu.sync_copy(cnt_hbm.at[pl.ds(pl.multiple_of(w * L, 8), L)], cntv)
    total = cntv[pl.ds(0, L)][0]
    iota = lax.iota(jnp.int32, L)
    sidxs = (sidx0, sidx1)
    didxs = (didx0, didx1)
    gsems = (gsem0, gsem1)
    ssems = (ssem0, ssem1)

    def stage_async(j, b):
        off = pl.multiple_of(wbase + j * CH, 8)
        pltpu.async_copy(esrc_hbm.at[pl.ds(off, CH)], sidxs[b], ssems[b])
        pltpu.async_copy(edloc_hbm.at[pl.ds(off, CH)], didxs[b], ssems[b])

    def wait_stage(b):
        pltpu.make_async_copy(esrc_hbm.at[pl.ds(0, CH)], sidxs[b],
                              ssems[b]).wait()
        pltpu.make_async_copy(edloc_hbm.at[pl.ds(0, CH)], didxs[b],
                              ssems[b]).wait()

    def clamp_launch_gather(b):
        for g in range(CH // L):
            sv = sidxs[b][pl.ds(g * L, L)]
            sv = jnp.where(sv < 0, 0, sv)
            sidxs[b][pl.ds(g * L, L)] = jnp.where(sv >= N, 0, sv)
        pltpu.async_copy(hs_hbm.at[sidxs[b]], rows.at[b], gsems[b])

    def wait_gather(b):
        pltpu.make_async_copy(hs_hbm.at[sidxs[b]], rows.at[b],
                              gsems[b]).wait()

    stage_async(0, 0)
    wait_stage(0)
    clamp_launch_gather(0)
    stage_async(1, 1)

    M = jnp.maximum((total + CH - 1) // CH, 1)

    def pair(k, carry):
        for b in range(2):
            j = 2 * k + b
            b2 = 1 - b
            wait_gather(b)
            for g in range(CH // L):
                dtmp[pl.ds(g * L, L)] = didxs[b][pl.ds(g * L, L)]
            wait_stage(b2)
            clamp_launch_gather(b2)
            stage_async(j + 2, b)

            @plsc.parallel_loop(0, CH // L, unroll=2)
            def group(g):
                dvec = dtmp[pl.ds(g * L, L)]
                lane = j * CH + g * L + iota
                ok = (lane < total) & (dvec >= 0) & (dvec < PT)
                dv = jnp.where(ok, dvec, TRASH_R)
                for l in range(L):
                    e = g * L + l
                    vals = [rows[b, e, pl.ds(kk * L, L)]
                            for kk in range(D // L)]
                    for kk in range(D // L):
                        plsc.addupdate(acc.at[dv[l], pl.ds(kk * L, L)],
                                       vals[kk])
        return carry

    lax.fori_loop(0, (M + 1) // 2, pair, 0, unroll=False)
    wait_gather(0)
    wait_stage(1)
    pltpu.sync_copy(acc, out_hbm.at[pl.ds(w * ACC_R, ACC_R)])


@functools.partial(
    pl.kernel,
    out_type=jax.ShapeDtypeStruct((VPAD, D), jnp.float32),
    mesh=_sc_mesh,
    scratch_types=[
        pltpu.VMEM((BPW,), jnp.int32),
        pltpu.VMEM((BPW, D), jnp.float32),
        pltpu.SemaphoreType.DMA,
    ],
)
def _gather_kernel(final_hbm, idx_hbm, out_hbm, idx_v, rows_v, sem):
    base = _wid() * BPW
    pltpu.sync_copy(idx_hbm.at[pl.ds(base, BPW)], idx_v)
    for k in range(BPW // 128):
        pltpu.async_copy(final_hbm.at[idx_v.at[pl.ds(k * 128, 128)]],
                         rows_v.at[pl.ds(k * 128, 128)], sem).wait()
    pltpu.sync_copy(rows_v, out_hbm.at[pl.ds(base, BPW)])


RB = 1000


def _dinv(deg_ref):
    return lax.rsqrt(deg_ref[:, 0:1] + 1.0)


def _mm_scale_body(deg_ref, x_ref, w_ref, out_ref):
    out_ref[...] = jnp.dot(
        x_ref[...], w_ref[...],
        preferred_element_type=jnp.float32) * _dinv(deg_ref)


def _mid_body(deg_ref, agg_ref, hs_ref, b_ref, w_ref, out_ref):
    dinv = _dinv(deg_ref)
    x = jnp.maximum(dinv * (agg_ref[...] + hs_ref[...]) + b_ref[...], 0.0)
    out_ref[...] = jnp.dot(
        x, w_ref[...], preferred_element_type=jnp.float32) * dinv


def _final_body(deg_ref, agg_ref, hs_ref, b_ref, out_ref):
    dinv = _dinv(deg_ref)
    out_ref[...] = dinv * (agg_ref[...] + hs_ref[...]) + b_ref[...]


_deg_spec = pl.BlockSpec((RB, L), lambda i: (i, 0))
_row_spec = pl.BlockSpec((RB, D), lambda i: (i, 0))
_w_spec = pl.BlockSpec((D, D), lambda i: (0, 0))
_b_spec = pl.BlockSpec((1, D), lambda i: (0, 0))
_out_sds = jax.ShapeDtypeStruct((N, D), jnp.float32)

_mm_scale = pl.pallas_call(
    _mm_scale_body, grid=(N // RB,),
    in_specs=[_deg_spec, _row_spec, _w_spec],
    out_specs=_row_spec, out_shape=_out_sds)

_mid = pl.pallas_call(
    _mid_body, grid=(N // RB,),
    in_specs=[_deg_spec, _row_spec, _row_spec, _b_spec, _w_spec],
    out_specs=_row_spec, out_shape=_out_sds)

_final = pl.pallas_call(
    _final_body, grid=(N // RB,),
    in_specs=[_deg_spec, _row_spec, _row_spec, _b_spec],
    out_specs=_row_spec, out_shape=_out_sds)


def _unzone(x):
    dcols = x.shape[-1]
    return x.reshape(W, ACC_R, dcols)[:, :PT].reshape(W * PT, dcols)[:N]


def kernel(embedding, edge_index, W1, b1, W2, b2, idx_mapping):
    src = edge_index[0]
    dst = edge_index[1]
    zeros_agg = jnp.zeros((ACC_R, D), jnp.float32)

    esrc, edloc, counts = _route_kernel(src, dst)
    deg = _unzone(_deg_kernel(edloc, counts))

    hs1 = _mm_scale(deg, embedding, W1)
    agg1 = _unzone(_agg_kernel(esrc, edloc, counts, hs1, zeros_agg))
    hs2 = _mid(deg, agg1, hs1, b1.reshape(1, D), W2)
    agg2 = _unzone(_agg_kernel(esrc, edloc, counts, hs2, zeros_agg))
    finaln = _final(deg, agg2, hs2, b2.reshape(1, D))

    idx_pad = jnp.concatenate(
        [idx_mapping, jnp.zeros((VPAD - V,), jnp.int32)])
    out = _gather_kernel(finaln, idx_pad)
    return out[:V]
